# baseline (speedup 1.0000x reference)
import functools

import jax
import jax.numpy as jnp
from jax import lax
from jax.experimental import layout as jax_layout
from jax.experimental import pallas as pl
from jax.experimental.pallas import tpu as pltpu
from jax.experimental.pallas import tpu_sc as plsc

B = 4096
L = 50
D = 64

NUM_CORES = 2
NUM_SUBCORES = 16
NW = NUM_CORES * NUM_SUBCORES
BPW = B // NW
PPW = BPW // 2
LANES = 16
DC = D // LANES

NBUF = 4
NGRP = PPW // NBUF


def _bag_sum(rows_v, p, off):
    def body(r, accs):
        base = off + r * 2
        new = []
        for dc in range(DC):
            sl = pl.ds(dc * LANES, LANES)
            a = rows_v[p, 0, base, sl] + rows_v[p, 0, base + 1, sl]
            b = rows_v[p, 1, base, sl] + rows_v[p, 1, base + 1, sl]
            new.append(accs[dc] + (a + b))
        return tuple(new)

    init = tuple(jnp.zeros((LANES,), jnp.float32) for _ in range(DC))
    return lax.fori_loop(0, L // 2, body, init, unroll=5)


def _issue_gathers(table_hbm, ids_v, rows_v, jp, p, sem):
    for f in range(2):
        pltpu.async_copy(table_hbm.at[ids_v.at[f, jp]], rows_v.at[p, f], sem)


def _wait_gathers(table_hbm, ids_v, rows_v, jp, p, sem):
    for f in range(2):
        pltpu.make_async_copy(
            table_hbm.at[ids_v.at[f, jp]], rows_v.at[p, f], sem).wait()


def _embed_body(ids_t_hbm, ids_d_hbm, table_hbm, out_hbm, ids_v, rows_v,
                out_v, *sems):
    wid = lax.axis_index("s") * NUM_CORES + lax.axis_index("c")
    base = wid * PPW
    pltpu.sync_copy(ids_t_hbm.at[pl.ds(base, PPW)], ids_v.at[0])
    pltpu.sync_copy(ids_d_hbm.at[pl.ds(base, PPW)], ids_v.at[1])

    for p in range(NBUF):
        _issue_gathers(table_hbm, ids_v, rows_v, p, p, sems[p])

    def group(g, _):
        for p in range(NBUF):
            jp = g * NBUF + p
            _wait_gathers(table_hbm, ids_v, rows_v, jp, p, sems[p])
            for h in range(2):
                accs = _bag_sum(rows_v, p, h * L)
                for dc in range(DC):
                    out_v[2 * jp + h, pl.ds(dc * LANES, LANES)] = accs[dc]

            @pl.when(g < NGRP - 1)
            def _():
                _issue_gathers(table_hbm, ids_v, rows_v, jp + NBUF, p,
                               sems[p])
        return 0

    lax.fori_loop(0, NGRP, group, 0, unroll=False)
    pltpu.sync_copy(out_v, out_hbm.at[pl.ds(wid * BPW, BPW)])


_mesh = plsc.VectorSubcoreMesh(core_axis_name="c", subcore_axis_name="s")

_embed = functools.partial(
    pl.kernel,
    out_type=jax.ShapeDtypeStruct((B, D), jnp.float32),
    mesh=_mesh,
    scratch_types=[
        pltpu.VMEM((2, PPW, 2 * L), jnp.int32),
        pltpu.VMEM((NBUF, 2, 2 * L, D), jnp.float32),
        pltpu.VMEM((BPW, D), jnp.float32),
    ] + [pltpu.SemaphoreType.DMA] * NBUF,
    compiler_params=pltpu.CompilerParams(use_tc_tiling_on_sc=False),
)(_embed_body)


def _run(table, title_input_ids, title_offsets, description_input_ids,
         description_offsets):
    del title_offsets, description_offsets
    return _embed(title_input_ids.reshape(B // 2, 2 * L),
                  description_input_ids.reshape(B // 2, 2 * L), table)


_jitted = None


def kernel(table, title_input_ids, title_offsets, description_input_ids,
           description_offsets):
    global _jitted
    if _jitted is None:
        sharding = jax.sharding.SingleDeviceSharding(jax.devices()[0])
        row_major = jax_layout.Format(
            jax_layout.Layout(major_to_minor=(0, 1), tiling=((8, 128),)),
            sharding)
        _jitted = jax.jit(
            _run, in_shardings=(row_major, None, None, None, None))
    return _jitted(table, title_input_ids, title_offsets,
                   description_input_ids, description_offsets)
